# Initial kernel scaffold; baseline (speedup 1.0000x reference)
import jax
import jax.numpy as jnp
from jax.experimental import pallas as pl


def kernel(heads, rels, tails, embed_e, embed_r):
    raise NotImplementedError("write your pallas kernel here")


# speedup vs baseline: 2.0467x; 2.0467x over previous
import jax
import jax.numpy as jnp
from jax import lax
from jax.experimental import pallas as pl
from jax.experimental.pallas import tpu as pltpu
from jax.experimental.pallas import tpu_sc as plsc

NC = 2
NS = 16
L = 16
NW = NC * NS

BATCH = 16384
K = 50
DPAD = 128
BPW = BATCH // NW
CHUNK = 128
NCHUNK = BPW // CHUNK
NSLOT = 2


def _sc_body(heads_hbm, rels_hbm, tails_hbm, embed_e_hbm, embed_r_hbm,
             out_hbm, idx_h, idx_r, idx_t, rows_h, rows_r, rows_t,
             acc_v, sem0, sem1):
    wid = lax.axis_index("s") * NC + lax.axis_index("c")
    sems = (sem0, sem1)

    pltpu.sync_copy(heads_hbm.at[wid], idx_h)
    pltpu.sync_copy(rels_hbm.at[wid], idx_r)
    pltpu.sync_copy(tails_hbm.at[wid], idx_t)

    def chunk_copies(j):
        s = j % NSLOT
        return [
            pltpu.make_async_copy(embed_e_hbm.at[idx_h.at[j]],
                                  rows_h.at[s], sems[s]),
            pltpu.make_async_copy(embed_r_hbm.at[idx_r.at[j]],
                                  rows_r.at[s], sems[s]),
            pltpu.make_async_copy(embed_e_hbm.at[idx_t.at[j]],
                                  rows_t.at[s], sems[s]),
        ]

    zero = jnp.zeros((L,), jnp.float32)

    for c in chunk_copies(0):
        c.start()

    acc = zero
    for j in range(NCHUNK):
        s = j % NSLOT
        for c in chunk_copies(j):
            c.wait()
        if j + 1 < NCHUNK:
            for c in chunk_copies(j + 1):
                c.start()

        def body(i, a, s=s):
            for c0 in (0, L, 2 * L, 3 * L):
                d = (rows_h[s, i, pl.ds(c0, L)] + rows_r[s, i, pl.ds(c0, L)]
                     - rows_t[s, i, pl.ds(c0, L)])
                a = a + jnp.abs(d)
            return a

        acc = lax.fori_loop(0, CHUNK, body, acc)

    acc_v[...] = acc
    pltpu.sync_copy(acc_v, out_hbm.at[wid])


@jax.jit
def kernel(heads, rels, tails, embed_e, embed_r):
    heads_r = heads.astype(jnp.int32).reshape(NW, NCHUNK, CHUNK)
    rels_r = rels.astype(jnp.int32).reshape(NW, NCHUNK, CHUNK)
    tails_r = tails.astype(jnp.int32).reshape(NW, NCHUNK, CHUNK)
    embed_e_p = jnp.pad(embed_e, ((0, 0), (0, DPAD - K)))
    embed_r_p = jnp.pad(embed_r, ((0, 0), (0, DPAD - K)))

    mesh = plsc.VectorSubcoreMesh(
        core_axis_name="c", subcore_axis_name="s",
        num_cores=NC, num_subcores=NS)
    run = pl.kernel(
        _sc_body,
        out_type=jax.ShapeDtypeStruct((NW, L), jnp.float32),
        mesh=mesh,
        scratch_types=[
            pltpu.VMEM((NCHUNK, CHUNK), jnp.int32),
            pltpu.VMEM((NCHUNK, CHUNK), jnp.int32),
            pltpu.VMEM((NCHUNK, CHUNK), jnp.int32),
            pltpu.VMEM((NSLOT, CHUNK, DPAD), jnp.float32),
            pltpu.VMEM((NSLOT, CHUNK, DPAD), jnp.float32),
            pltpu.VMEM((NSLOT, CHUNK, DPAD), jnp.float32),
            pltpu.VMEM((L,), jnp.float32),
            pltpu.SemaphoreType.DMA,
            pltpu.SemaphoreType.DMA,
        ],
    )
    partials = run(heads_r, rels_r, tails_r, embed_e_p, embed_r_p)
    return jnp.sum(partials)
